# Initial kernel scaffold; baseline (speedup 1.0000x reference)
import functools

import jax
import jax.numpy as jnp
from jax import lax
from jax.experimental import pallas as pl
from jax.experimental.pallas import tpu as pltpu
from jax.experimental.pallas import tpu_sc as plsc

N = 10000
E = 160000
D = 256
H = 256
C = 64

NC = 2
NS = 16
LANES = 16

BN = 400
NB = N // BN
RPT = N // NS
ZR = 125

_MESH = plsc.VectorSubcoreMesh(core_axis_name="c", subcore_axis_name="s")


def _zero_buf(buf, rows, width):
    zv = jnp.zeros((LANES,), jnp.float32)

    def body(r, _):
        for c in range(width // LANES):
            buf[r, pl.ds(c * LANES, LANES)] = zv
        return 0

    lax.fori_loop(0, rows, body, 0)


DEGW = 16
DEG_K = 40
DEG_EPT = E // (NC * NS)
DEG_CH = DEG_EPT // DEG_K


def _deg_body(dst_hbm, deg_hbm, idx_v, ones_v, zbuf, acc_sh):
    cid = lax.axis_index("c")
    sid = lax.axis_index("s")
    wid = cid * NS + sid

    ov = jnp.ones((LANES,), jnp.float32)

    def fill_ones(r, _):
        ones_v[r, pl.ds(0, LANES)] = ov
        return 0

    lax.fori_loop(0, DEG_K, fill_ones, 0)
    _zero_buf(zbuf, ZR, DEGW)
    for k in range(RPT // ZR):
        pltpu.sync_copy(zbuf, acc_sh.at[pl.ds(sid * RPT + k * ZR, ZR)])
    plsc.subcore_barrier()

    def body(j, _):
        base = wid * DEG_EPT + j * DEG_K
        pltpu.sync_copy(dst_hbm.at[pl.ds(base, DEG_K)], idx_v)
        pltpu.sync_copy(ones_v, acc_sh.at[idx_v], add=True)
        return 0

    lax.fori_loop(0, DEG_CH, body, 0)
    plsc.subcore_barrier()

    for k in range(RPT // ZR):
        r0 = sid * RPT + k * ZR
        pltpu.sync_copy(acc_sh.at[pl.ds(r0, ZR)], zbuf)
        pltpu.sync_copy(zbuf, deg_hbm.at[pl.ds(cid * N + r0, ZR)])


_deg_kernel = functools.partial(
    pl.kernel,
    out_type=jax.ShapeDtypeStruct((NC * N, DEGW), jnp.float32),
    mesh=_MESH,
    scratch_types=[
        pltpu.VMEM((DEG_K,), jnp.int32),
        pltpu.VMEM((DEG_K, DEGW), jnp.float32),
        pltpu.VMEM((ZR, DEGW), jnp.float32),
        pltpu.VMEM_SHARED((N, DEGW), jnp.float32),
    ],
)(_deg_body)


SCAT_K = 80
SCAT_EPT = E // NS
SCAT_CH = SCAT_EPT // SCAT_K


def _scatter_body(W, g_hbm, src_hbm, dst_hbm, out_hbm,
                  src_v, dst_v, rows_v, zbuf, acc_sh, sem):
    cid = lax.axis_index("c")
    sid = lax.axis_index("s")
    off = cid * N

    _zero_buf(zbuf, ZR, W)
    for k in range(RPT // ZR):
        pltpu.sync_copy(zbuf, acc_sh.at[pl.ds(sid * RPT + k * ZR, ZR)])
    plsc.subcore_barrier()

    def body(j, _):
        base = sid * SCAT_EPT + j * SCAT_K
        pltpu.sync_copy(src_hbm.at[pl.ds(base, SCAT_K)], src_v)
        pltpu.sync_copy(dst_hbm.at[pl.ds(base, SCAT_K)], dst_v)
        for r in range(SCAT_K // LANES):
            sl = pl.ds(r * LANES, LANES)
            src_v[sl] = src_v[sl] + off
        pltpu.async_copy(g_hbm.at[src_v], rows_v, sem).wait()
        pltpu.sync_copy(rows_v, acc_sh.at[dst_v], add=True)
        return 0

    lax.fori_loop(0, SCAT_CH, body, 0)
    plsc.subcore_barrier()

    for k in range(RPT // ZR):
        r0 = sid * RPT + k * ZR
        pltpu.sync_copy(acc_sh.at[pl.ds(r0, ZR)], zbuf)
        pltpu.sync_copy(zbuf, out_hbm.at[pl.ds(off + r0, ZR)])


def _make_scatter(W):
    return functools.partial(
        pl.kernel,
        out_type=jax.ShapeDtypeStruct((NC * N, W), jnp.float32),
        mesh=_MESH,
        scratch_types=[
            pltpu.VMEM((SCAT_K,), jnp.int32),
            pltpu.VMEM((SCAT_K,), jnp.int32),
            pltpu.VMEM((SCAT_K, W), jnp.float32),
            pltpu.VMEM((ZR, W), jnp.float32),
            pltpu.VMEM_SHARED((N, W), jnp.float32),
            pltpu.SemaphoreType.DMA,
        ],
    )(functools.partial(_scatter_body, W))


_scatter128 = _make_scatter(H // NC)
_scatter32 = _make_scatter(C // NC)


def _dinv_block(deg_a, deg_b):
    return lax.rsqrt(deg_a[:, 0:1] + deg_b[:, 0:1] + 1.0)


def _mm1_body(x_ref, w_ref, dga_ref, dgb_ref, out_ref):
    dinv = _dinv_block(dga_ref[...], dgb_ref[...])
    h = jnp.dot(x_ref[...], w_ref[...], preferred_element_type=jnp.float32)
    out_ref[...] = dinv * h


_mm1 = pl.pallas_call(
    _mm1_body,
    grid=(NC, NB),
    in_specs=[
        pl.BlockSpec((BN, D), lambda p, i: (i, 0)),
        pl.BlockSpec((D, H // NC), lambda p, i: (0, p)),
        pl.BlockSpec((BN, DEGW), lambda p, i: (i, 0)),
        pl.BlockSpec((BN, DEGW), lambda p, i: (NB + i, 0)),
    ],
    out_specs=pl.BlockSpec((BN, H // NC), lambda p, i: (p * NB + i, 0)),
    out_shape=jax.ShapeDtypeStruct((NC * N, H // NC), jnp.float32),
)


def _mm2_body(s1a_ref, s1b_ref, g1a_ref, g1b_ref, w2_ref, b1_ref,
              dga_ref, dgb_ref, out_ref):
    dinv = _dinv_block(dga_ref[...], dgb_ref[...])
    x1a = jax.nn.relu(dinv * (s1a_ref[...] + g1a_ref[...]) + b1_ref[0:1, 0:128])
    x1b = jax.nn.relu(dinv * (s1b_ref[...] + g1b_ref[...]) + b1_ref[0:1, 128:256])
    acc = jnp.dot(x1a, w2_ref[0:128, :], preferred_element_type=jnp.float32)
    acc += jnp.dot(x1b, w2_ref[128:256, :], preferred_element_type=jnp.float32)
    out_ref[...] = dinv * acc


_mm2 = pl.pallas_call(
    _mm2_body,
    grid=(NC, NB),
    in_specs=[
        pl.BlockSpec((BN, H // NC), lambda p, i: (i, 0)),
        pl.BlockSpec((BN, H // NC), lambda p, i: (NB + i, 0)),
        pl.BlockSpec((BN, H // NC), lambda p, i: (i, 0)),
        pl.BlockSpec((BN, H // NC), lambda p, i: (NB + i, 0)),
        pl.BlockSpec((H, C // NC), lambda p, i: (0, p)),
        pl.BlockSpec((1, H), lambda p, i: (0, 0)),
        pl.BlockSpec((BN, DEGW), lambda p, i: (i, 0)),
        pl.BlockSpec((BN, DEGW), lambda p, i: (NB + i, 0)),
    ],
    out_specs=pl.BlockSpec((BN, C // NC), lambda p, i: (p * NB + i, 0)),
    out_shape=jax.ShapeDtypeStruct((NC * N, C // NC), jnp.float32),
)


def _fin_body(s2a_ref, s2b_ref, g2a_ref, g2b_ref, b2_ref,
              dga_ref, dgb_ref, out_ref):
    dinv = _dinv_block(dga_ref[...], dgb_ref[...])
    ha = dinv * (s2a_ref[...] + g2a_ref[...]) + b2_ref[0:1, 0:32]
    hb = dinv * (s2b_ref[...] + g2b_ref[...]) + b2_ref[0:1, 32:64]
    out_ref[...] = jnp.concatenate([ha, hb], axis=1)


_fin = pl.pallas_call(
    _fin_body,
    grid=(NB,),
    in_specs=[
        pl.BlockSpec((BN, C // NC), lambda i: (i, 0)),
        pl.BlockSpec((BN, C // NC), lambda i: (NB + i, 0)),
        pl.BlockSpec((BN, C // NC), lambda i: (i, 0)),
        pl.BlockSpec((BN, C // NC), lambda i: (NB + i, 0)),
        pl.BlockSpec((1, C), lambda i: (0, 0)),
        pl.BlockSpec((BN, DEGW), lambda i: (i, 0)),
        pl.BlockSpec((BN, DEGW), lambda i: (NB + i, 0)),
    ],
    out_specs=pl.BlockSpec((BN, C), lambda i: (i, 0)),
    out_shape=jax.ShapeDtypeStruct((N, C), jnp.float32),
)


def kernel(last_e_emb, edge_index, W1, b1, W2, b2):
    src = edge_index[0]
    dst = edge_index[1]
    deg = _deg_kernel(dst)
    g1 = _mm1(last_e_emb, W1, deg, deg)
    s1 = _scatter128(g1, src, dst)
    g2 = _mm2(s1, s1, g1, g1, W2, b1.reshape(1, H), deg, deg)
    s2 = _scatter32(g2, src, dst)
    return _fin(s2, s2, g2, g2, b2.reshape(1, C), deg, deg)


# speedup vs baseline: 5.2850x; 5.2850x over previous
import functools

import jax
import jax.numpy as jnp
from jax import lax
from jax.experimental import pallas as pl
from jax.experimental.pallas import tpu as pltpu
from jax.experimental.pallas import tpu_sc as plsc

N = 10000
E = 160000
D = 256
H = 256
C = 64

NC = 2
NS = 16
LANES = 16

NP = 10240
BN = 80
NB = N // BN
NPB = NP // BN
RPT = NP // NS
ZR = 160

_MESH = plsc.VectorSubcoreMesh(core_axis_name="c", subcore_axis_name="s")


def _zero_buf(buf, rows, width):
    zv = jnp.zeros((LANES,), jnp.float32)

    def body(r, _):
        for c in range(width // LANES):
            buf[r, pl.ds(c * LANES, LANES)] = zv
        return 0

    lax.fori_loop(0, rows, body, 0)


DEGW = 16
DEG_K = 40
DEG_EPT = E // (NC * NS)
DEG_CH = DEG_EPT // DEG_K


def _deg_body(dst_hbm, deg_hbm, idx_v, ones_v, zbuf, acc_sh):
    cid = lax.axis_index("c")
    sid = lax.axis_index("s")
    wid = cid * NS + sid

    ov = jnp.ones((LANES,), jnp.float32)

    def fill_ones(r, _):
        ones_v[r, pl.ds(0, LANES)] = ov
        return 0

    lax.fori_loop(0, DEG_K, fill_ones, 0)
    _zero_buf(zbuf, ZR, DEGW)
    for k in range(RPT // ZR):
        pltpu.sync_copy(zbuf, acc_sh.at[pl.ds(sid * RPT + k * ZR, ZR)])
    plsc.subcore_barrier()

    def body(j, _):
        base = wid * DEG_EPT + j * DEG_K
        pltpu.sync_copy(dst_hbm.at[pl.ds(base, DEG_K)], idx_v)
        pltpu.sync_copy(ones_v, acc_sh.at[idx_v], add=True)
        return 0

    lax.fori_loop(0, DEG_CH, body, 0)
    plsc.subcore_barrier()

    for k in range(RPT // ZR):
        r0 = sid * RPT + k * ZR
        pltpu.sync_copy(acc_sh.at[pl.ds(r0, ZR)], zbuf)
        pltpu.sync_copy(zbuf, deg_hbm.at[pl.ds(cid * NP + r0, ZR)])


_SC_PARAMS = pltpu.CompilerParams(use_tc_tiling_on_sc=False)

_deg_kernel = functools.partial(
    pl.kernel,
    out_type=jax.ShapeDtypeStruct((NC * NP, DEGW), jnp.float32),
    mesh=_MESH,
    compiler_params=_SC_PARAMS,
    scratch_types=[
        pltpu.VMEM((DEG_K,), jnp.int32),
        pltpu.VMEM((DEG_K, DEGW), jnp.float32),
        pltpu.VMEM((ZR, DEGW), jnp.float32),
        pltpu.VMEM_SHARED((NP, DEGW), jnp.float32),
    ],
)(_deg_body)


SCAT_K = 80
SCAT_EPT = E // NS
SCAT_CH = SCAT_EPT // SCAT_K


def _scatter_body(W, g_hbm, src_hbm, dst_hbm, out_hbm,
                  src_v, dst_v, rows_v, zbuf, acc_sh, sem):
    cid = lax.axis_index("c")
    sid = lax.axis_index("s")
    off = cid * NP

    _zero_buf(zbuf, ZR, W)
    for k in range(RPT // ZR):
        pltpu.sync_copy(zbuf, acc_sh.at[pl.ds(sid * RPT + k * ZR, ZR)])
    plsc.subcore_barrier()

    def body(j, _):
        base = sid * SCAT_EPT + j * SCAT_K
        pltpu.sync_copy(src_hbm.at[pl.ds(base, SCAT_K)], src_v)
        pltpu.sync_copy(dst_hbm.at[pl.ds(base, SCAT_K)], dst_v)
        for r in range(SCAT_K // LANES):
            sl = pl.ds(r * LANES, LANES)
            src_v[sl] = src_v[sl] + off
        pltpu.async_copy(g_hbm.at[src_v], rows_v, sem).wait()
        pltpu.sync_copy(rows_v, acc_sh.at[dst_v], add=True)
        return 0

    lax.fori_loop(0, SCAT_CH, body, 0)
    plsc.subcore_barrier()

    for k in range(RPT // ZR):
        r0 = sid * RPT + k * ZR
        pltpu.sync_copy(acc_sh.at[pl.ds(r0, ZR)], zbuf)
        pltpu.sync_copy(zbuf, out_hbm.at[pl.ds(off + r0, ZR)])


def _make_scatter(W):
    return functools.partial(
        pl.kernel,
        out_type=jax.ShapeDtypeStruct((NC * NP, W), jnp.float32),
        mesh=_MESH,
        compiler_params=_SC_PARAMS,
        scratch_types=[
            pltpu.VMEM((SCAT_K,), jnp.int32),
            pltpu.VMEM((SCAT_K,), jnp.int32),
            pltpu.VMEM((SCAT_K, W), jnp.float32),
            pltpu.VMEM((ZR, W), jnp.float32),
            pltpu.VMEM_SHARED((NP, W), jnp.float32),
            pltpu.SemaphoreType.DMA,
        ],
    )(functools.partial(_scatter_body, W))


_scatter128 = _make_scatter(H // NC)
_scatter32 = _make_scatter(C // NC)


def _dinv_block(deg_a, deg_b):
    return lax.rsqrt(deg_a[:, 0:1] + deg_b[:, 0:1] + 1.0)


def _mm1_body(x_ref, w_ref, dga_ref, dgb_ref, out_ref):
    dinv = _dinv_block(dga_ref[...], dgb_ref[...])
    h = jnp.dot(x_ref[...], w_ref[...], preferred_element_type=jnp.float32)
    out_ref[...] = dinv * h


_mm1 = pl.pallas_call(
    _mm1_body,
    grid=(NC, NB),
    in_specs=[
        pl.BlockSpec((BN, D), lambda p, i: (i, 0)),
        pl.BlockSpec((D, H // NC), lambda p, i: (0, p)),
        pl.BlockSpec((BN, DEGW), lambda p, i: (i, 0)),
        pl.BlockSpec((BN, DEGW), lambda p, i: (NPB + i, 0)),
    ],
    out_specs=pl.BlockSpec((BN, H // NC), lambda p, i: (p * NPB + i, 0)),
    out_shape=jax.ShapeDtypeStruct((NC * NP, H // NC), jnp.float32),
)


def _mm2_body(s1a_ref, s1b_ref, g1a_ref, g1b_ref, w2t_ref, b1_ref,
              dga_ref, dgb_ref, out_ref):
    dinv = _dinv_block(dga_ref[...], dgb_ref[...])
    x1a = jax.nn.relu(dinv * (s1a_ref[...] + g1a_ref[...]) + b1_ref[0:1, 0:128])
    x1b = jax.nn.relu(dinv * (s1b_ref[...] + g1b_ref[...]) + b1_ref[0:1, 128:256])
    dn = (((1,), (1,)), ((), ()))
    acc = lax.dot_general(x1a, w2t_ref[:, 0:128], dn,
                          preferred_element_type=jnp.float32)
    acc += lax.dot_general(x1b, w2t_ref[:, 128:256], dn,
                           preferred_element_type=jnp.float32)
    out_ref[...] = dinv * acc


_mm2 = pl.pallas_call(
    _mm2_body,
    grid=(NC, NB),
    in_specs=[
        pl.BlockSpec((BN, H // NC), lambda p, i: (i, 0)),
        pl.BlockSpec((BN, H // NC), lambda p, i: (NPB + i, 0)),
        pl.BlockSpec((BN, H // NC), lambda p, i: (i, 0)),
        pl.BlockSpec((BN, H // NC), lambda p, i: (NPB + i, 0)),
        pl.BlockSpec((C // NC, H), lambda p, i: (p, 0)),
        pl.BlockSpec((1, H), lambda p, i: (0, 0)),
        pl.BlockSpec((BN, DEGW), lambda p, i: (i, 0)),
        pl.BlockSpec((BN, DEGW), lambda p, i: (NPB + i, 0)),
    ],
    out_specs=pl.BlockSpec((BN, C // NC), lambda p, i: (p * NPB + i, 0)),
    out_shape=jax.ShapeDtypeStruct((NC * NP, C // NC), jnp.float32),
)


def _fin_body(s2a_ref, s2b_ref, g2a_ref, g2b_ref, b2_ref,
              dga_ref, dgb_ref, out_ref):
    dinv = _dinv_block(dga_ref[...], dgb_ref[...])
    ha = dinv * (s2a_ref[...] + g2a_ref[...]) + b2_ref[0:1, 0:32]
    hb = dinv * (s2b_ref[...] + g2b_ref[...]) + b2_ref[0:1, 32:64]
    out_ref[...] = jnp.concatenate([ha, hb], axis=1)


_fin = pl.pallas_call(
    _fin_body,
    grid=(NB,),
    in_specs=[
        pl.BlockSpec((BN, C // NC), lambda i: (i, 0)),
        pl.BlockSpec((BN, C // NC), lambda i: (NPB + i, 0)),
        pl.BlockSpec((BN, C // NC), lambda i: (i, 0)),
        pl.BlockSpec((BN, C // NC), lambda i: (NPB + i, 0)),
        pl.BlockSpec((1, C), lambda i: (0, 0)),
        pl.BlockSpec((BN, DEGW), lambda i: (i, 0)),
        pl.BlockSpec((BN, DEGW), lambda i: (NPB + i, 0)),
    ],
    out_specs=pl.BlockSpec((BN, C), lambda i: (i, 0)),
    out_shape=jax.ShapeDtypeStruct((N, C), jnp.float32),
)


def kernel(last_e_emb, edge_index, W1, b1, W2, b2):
    src = edge_index[0]
    dst = edge_index[1]
    deg = _deg_kernel(dst)
    g1 = _mm1(last_e_emb, W1, deg, deg)
    s1 = _scatter128(g1, src, dst)
    g2 = _mm2(s1, s1, g1, g1, W2.T, b1.reshape(1, H), deg, deg)
    s2 = _scatter32(g2, src, dst)
    return _fin(s2, s2, g2, g2, b2.reshape(1, C), deg, deg)


# speedup vs baseline: 5.8051x; 1.0984x over previous
import functools

import jax
import jax.numpy as jnp
from jax import lax
from jax.experimental import pallas as pl
from jax.experimental.pallas import tpu as pltpu
from jax.experimental.pallas import tpu_sc as plsc

N = 10000
E = 160000
D = 256
H = 256
C = 64

NC = 2
NS = 16
LANES = 16

NP = 10240
BN = 80
NB = N // BN
NPB = NP // BN
RPT = NP // NS
ZR = 160

_MESH = plsc.VectorSubcoreMesh(core_axis_name="c", subcore_axis_name="s")


def _zero_buf(buf, rows, width):
    zv = jnp.zeros((LANES,), jnp.float32)

    def body(r, _):
        for c in range(width // LANES):
            buf[r, pl.ds(c * LANES, LANES)] = zv
        return 0

    lax.fori_loop(0, rows, body, 0)


DEG_K = 40
DEG_EPT = E // (NC * NS)
DEG_CH = DEG_EPT // DEG_K


NBUF = 5


def _deg_body(dst_hbm, deg_hbm, ones_v, zbuf, acc_sh, *ring):
    idx_v = ring[0:NBUF]
    ssem = ring[NBUF:2 * NBUF]
    cid = lax.axis_index("c")
    sid = lax.axis_index("s")
    wid = cid * NS + sid

    ov = jnp.ones((LANES,), jnp.float32)
    for o in (0, 16, 24):
        ones_v[pl.ds(o, LANES)] = ov
    zv = jnp.zeros((LANES,), jnp.float32)
    for o in range(0, RPT, LANES):
        zbuf[pl.ds(o, LANES)] = zv
    pltpu.sync_copy(zbuf, acc_sh.at[pl.ds(sid * RPT, RPT)])
    plsc.subcore_barrier()

    def body(j2, _):
        for b in range(NBUF):
            j = j2 * NBUF + b

            @pl.when(j >= NBUF)
            def _():
                pltpu.make_async_copy(ones_v, acc_sh.at[idx_v[b]], ssem[b]).wait()

            pltpu.sync_copy(dst_hbm.at[pl.ds(wid * DEG_EPT + j * DEG_K, DEG_K)],
                            idx_v[b])
            pltpu.async_copy(ones_v, acc_sh.at[idx_v[b]], ssem[b], add=True)
        return 0

    lax.fori_loop(0, DEG_CH // NBUF, body, 0)
    for b in range(NBUF):
        pltpu.make_async_copy(ones_v, acc_sh.at[idx_v[b]], ssem[b]).wait()
    plsc.subcore_barrier()

    r0 = sid * RPT
    pltpu.sync_copy(acc_sh.at[pl.ds(r0, RPT)], zbuf)
    pltpu.sync_copy(zbuf, deg_hbm.at[pl.ds(cid * NP + r0, RPT)])


_SC_PARAMS = pltpu.CompilerParams(use_tc_tiling_on_sc=False)

_deg_kernel = functools.partial(
    pl.kernel,
    out_type=jax.ShapeDtypeStruct((NC * NP,), jnp.float32),
    mesh=_MESH,
    compiler_params=_SC_PARAMS,
    scratch_types=(
        [pltpu.VMEM((DEG_K,), jnp.float32),
         pltpu.VMEM((RPT,), jnp.float32),
         pltpu.VMEM_SHARED((NP,), jnp.float32)]
        + [pltpu.VMEM((DEG_K,), jnp.int32) for _ in range(NBUF)]
        + [pltpu.SemaphoreType.DMA for _ in range(NBUF)]
    ),
)(_deg_body)


SCAT_K = 40
SCAT_EPT = E // NS
SCAT_CH = SCAT_EPT // SCAT_K
ZRS = 80


LOOK = 2


def _scatter_body(W, g_hbm, src_hbm, dst_hbm, out_hbm, zbuf, raw_v, acc_sh,
                  *ring):
    src_v = ring[0:NBUF]
    dst_v = ring[NBUF:2 * NBUF]
    rows_v = ring[2 * NBUF:3 * NBUF]
    gsem = ring[3 * NBUF:4 * NBUF]
    ssem = ring[4 * NBUF:5 * NBUF]
    cid = lax.axis_index("c")
    sid = lax.axis_index("s")
    off = cid * NP

    _zero_buf(zbuf, ZRS, W)
    for k in range(RPT // ZRS):
        pltpu.sync_copy(zbuf, acc_sh.at[pl.ds(sid * RPT + k * ZRS, ZRS)])
    plsc.subcore_barrier()

    def load_and_gather(j, b):
        base = sid * SCAT_EPT + j * SCAT_K
        pltpu.sync_copy(src_hbm.at[pl.ds(base, SCAT_K)], raw_v)
        pltpu.sync_copy(dst_hbm.at[pl.ds(base, SCAT_K)], dst_v[b])
        for o in (0, 16, 24):
            sl = pl.ds(o, LANES)
            src_v[b][sl] = raw_v[sl] + off
        pltpu.async_copy(g_hbm.at[src_v[b]], rows_v[b], gsem[b])

    for j in range(LOOK):
        load_and_gather(j, j)

    def body(j2, _):
        for b in range(NBUF):
            j = j2 * NBUF + b
            bi = (b + LOOK) % NBUF
            ji = j + LOOK

            @pl.when(ji - NBUF >= 0)
            def _():
                pltpu.make_async_copy(
                    rows_v[bi], acc_sh.at[dst_v[bi]], ssem[bi]).wait()

            @pl.when(ji < SCAT_CH)
            def _():
                load_and_gather(ji, bi)

            pltpu.make_async_copy(g_hbm.at[src_v[b]], rows_v[b], gsem[b]).wait()
            pltpu.async_copy(rows_v[b], acc_sh.at[dst_v[b]], ssem[b], add=True)
        return 0

    lax.fori_loop(0, SCAT_CH // NBUF, body, 0)
    for c in range(SCAT_CH - NBUF + LOOK, SCAT_CH):
        b = c % NBUF
        pltpu.make_async_copy(rows_v[b], acc_sh.at[dst_v[b]], ssem[b]).wait()
    plsc.subcore_barrier()

    for k in range(RPT // ZRS):
        r0 = sid * RPT + k * ZRS
        pltpu.sync_copy(acc_sh.at[pl.ds(r0, ZRS)], zbuf)
        pltpu.sync_copy(zbuf, out_hbm.at[pl.ds(off + r0, ZRS)])


def _make_scatter(W):
    return functools.partial(
        pl.kernel,
        out_type=jax.ShapeDtypeStruct((NC * NP, W), jnp.float32),
        mesh=_MESH,
        compiler_params=_SC_PARAMS,
        scratch_types=(
            [pltpu.VMEM((ZRS, W), jnp.float32),
             pltpu.VMEM((SCAT_K,), jnp.int32),
             pltpu.VMEM_SHARED((NP, W), jnp.float32)]
            + [pltpu.VMEM((SCAT_K,), jnp.int32) for _ in range(2 * NBUF)]
            + [pltpu.VMEM((SCAT_K, W), jnp.float32) for _ in range(NBUF)]
            + [pltpu.SemaphoreType.DMA for _ in range(2 * NBUF)]
        ),
    )(functools.partial(_scatter_body, W))


_scatter128 = _make_scatter(H // NC)
_scatter32 = _make_scatter(C // NC)


def _dinv_block(deg_a, deg_b):
    return lax.rsqrt(deg_a + deg_b + 1.0)


def _mm1_body(x_ref, w_ref, dga_ref, dgb_ref, out_ref):
    dinv = _dinv_block(dga_ref[...], dgb_ref[...])
    h = jnp.dot(x_ref[...], w_ref[...], preferred_element_type=jnp.float32)
    out_ref[...] = dinv * h


_mm1 = pl.pallas_call(
    _mm1_body,
    grid=(NC, NB),
    in_specs=[
        pl.BlockSpec((BN, D), lambda p, i: (i, 0)),
        pl.BlockSpec((D, H // NC), lambda p, i: (0, p)),
        pl.BlockSpec((BN, 1), lambda p, i: (i, 0)),
        pl.BlockSpec((BN, 1), lambda p, i: (NPB + i, 0)),
    ],
    out_specs=pl.BlockSpec((BN, H // NC), lambda p, i: (p * NPB + i, 0)),
    out_shape=jax.ShapeDtypeStruct((NC * NP, H // NC), jnp.float32),
)


def _mm2_body(s1a_ref, s1b_ref, g1a_ref, g1b_ref, w2t_ref, b1_ref,
              dga_ref, dgb_ref, out_ref):
    dinv = _dinv_block(dga_ref[...], dgb_ref[...])
    x1a = jax.nn.relu(dinv * (s1a_ref[...] + g1a_ref[...]) + b1_ref[0:1, 0:128])
    x1b = jax.nn.relu(dinv * (s1b_ref[...] + g1b_ref[...]) + b1_ref[0:1, 128:256])
    dn = (((1,), (1,)), ((), ()))
    acc = lax.dot_general(x1a, w2t_ref[:, 0:128], dn,
                          preferred_element_type=jnp.float32)
    acc += lax.dot_general(x1b, w2t_ref[:, 128:256], dn,
                           preferred_element_type=jnp.float32)
    out_ref[...] = dinv * acc


_mm2 = pl.pallas_call(
    _mm2_body,
    grid=(NC, NB),
    in_specs=[
        pl.BlockSpec((BN, H // NC), lambda p, i: (i, 0)),
        pl.BlockSpec((BN, H // NC), lambda p, i: (NPB + i, 0)),
        pl.BlockSpec((BN, H // NC), lambda p, i: (i, 0)),
        pl.BlockSpec((BN, H // NC), lambda p, i: (NPB + i, 0)),
        pl.BlockSpec((C // NC, H), lambda p, i: (p, 0)),
        pl.BlockSpec((1, H), lambda p, i: (0, 0)),
        pl.BlockSpec((BN, 1), lambda p, i: (i, 0)),
        pl.BlockSpec((BN, 1), lambda p, i: (NPB + i, 0)),
    ],
    out_specs=pl.BlockSpec((BN, C // NC), lambda p, i: (p * NPB + i, 0)),
    out_shape=jax.ShapeDtypeStruct((NC * NP, C // NC), jnp.float32),
)


def _fin_body(s2a_ref, s2b_ref, g2a_ref, g2b_ref, b2_ref,
              dga_ref, dgb_ref, out_ref):
    dinv = _dinv_block(dga_ref[...], dgb_ref[...])
    ha = dinv * (s2a_ref[...] + g2a_ref[...]) + b2_ref[0:1, 0:32]
    hb = dinv * (s2b_ref[...] + g2b_ref[...]) + b2_ref[0:1, 32:64]
    out_ref[...] = jnp.concatenate([ha, hb], axis=1)


_fin = pl.pallas_call(
    _fin_body,
    grid=(NB,),
    in_specs=[
        pl.BlockSpec((BN, C // NC), lambda i: (i, 0)),
        pl.BlockSpec((BN, C // NC), lambda i: (NPB + i, 0)),
        pl.BlockSpec((BN, C // NC), lambda i: (i, 0)),
        pl.BlockSpec((BN, C // NC), lambda i: (NPB + i, 0)),
        pl.BlockSpec((1, C), lambda i: (0, 0)),
        pl.BlockSpec((BN, 1), lambda i: (i, 0)),
        pl.BlockSpec((BN, 1), lambda i: (NPB + i, 0)),
    ],
    out_specs=pl.BlockSpec((BN, C), lambda i: (i, 0)),
    out_shape=jax.ShapeDtypeStruct((N, C), jnp.float32),
)


def kernel(last_e_emb, edge_index, W1, b1, W2, b2):
    src = edge_index[0]
    dst = edge_index[1]
    deg = _deg_kernel(dst).reshape(NC * NP, 1)
    g1 = _mm1(last_e_emb, W1, deg, deg)
    s1 = _scatter128(g1, src, dst)
    g2 = _mm2(s1, s1, g1, g1, W2.T, b1.reshape(1, H), deg, deg)
    s2 = _scatter32(g2, src, dst)
    return _fin(s2, s2, g2, g2, b2.reshape(1, C), deg, deg)


# speedup vs baseline: 13.1617x; 2.2673x over previous
import functools

import jax
import jax.numpy as jnp
from jax import lax
from jax.experimental import pallas as pl
from jax.experimental.pallas import tpu as pltpu
from jax.experimental.pallas import tpu_sc as plsc

N = 10000
E = 160000
D = 256
H = 256
C = 64

NC = 2
NS = 16
LANES = 16

BN = 400
NB = N // BN
RPT = N // NS
ZRS = 125
SEG = 624

_MESH = plsc.VectorSubcoreMesh(core_axis_name="c", subcore_axis_name="s")
_SC_PARAMS = pltpu.CompilerParams(use_tc_tiling_on_sc=False)


def _zero_buf(buf, rows, width):
    zv = jnp.zeros((LANES,), jnp.float32)

    def body(r, _):
        for c in range(width // LANES):
            buf[r, pl.ds(c * LANES, LANES)] = zv
        return 0

    lax.fori_loop(0, rows, body, 0)


DEG_K = 40
DEG_EPT = E // (NC * NS)
DEG_CH = DEG_EPT // DEG_K

NBUF = 5


def _deg_body(dst2_hbm, deg_hbm, ones_v, zbuf, idx2d, acc_sh, *ssem):
    cid = lax.axis_index("c")
    sid = lax.axis_index("s")
    wid = cid * NS + sid

    ov = jnp.ones((LANES,), jnp.float32)
    for o in (0, 16, 24):
        ones_v[pl.ds(o, LANES)] = ov
    zv = jnp.zeros((LANES,), jnp.float32)
    for o in range(0, SEG + LANES, LANES):
        zbuf[pl.ds(o, LANES)] = zv

    @pl.when(sid < NS - 1)
    def _():
        pltpu.sync_copy(zbuf.at[pl.ds(0, SEG)], acc_sh.at[pl.ds(sid * SEG, SEG)])

    @pl.when(sid == NS - 1)
    def _():
        pltpu.sync_copy(zbuf, acc_sh.at[pl.ds(sid * SEG, N - (NS - 1) * SEG)])

    pltpu.sync_copy(dst2_hbm.at[pl.ds(wid * DEG_CH, DEG_CH)], idx2d)
    plsc.subcore_barrier()

    def body(j2, _):
        for b in range(NBUF):
            j = j2 * NBUF + b

            @pl.when(j >= NBUF)
            def _():
                pltpu.make_async_copy(
                    ones_v, acc_sh.at[idx2d.at[j - NBUF]], ssem[b]).wait()

            pltpu.async_copy(ones_v, acc_sh.at[idx2d.at[j]], ssem[b], add=True)
        return 0

    lax.fori_loop(0, DEG_CH // NBUF, body, 0)
    for b in range(NBUF):
        pltpu.make_async_copy(ones_v, acc_sh.at[idx2d.at[0]], ssem[b]).wait()
    plsc.subcore_barrier()

    @pl.when(sid < NS - 1)
    def _():
        pltpu.sync_copy(acc_sh.at[pl.ds(sid * SEG, SEG)], zbuf.at[pl.ds(0, SEG)])
        pltpu.sync_copy(zbuf.at[pl.ds(0, SEG)],
                        deg_hbm.at[pl.ds(cid * N + sid * SEG, SEG)])

    @pl.when(sid == NS - 1)
    def _():
        tail = N - (NS - 1) * SEG
        pltpu.sync_copy(acc_sh.at[pl.ds(sid * SEG, tail)], zbuf)
        pltpu.sync_copy(zbuf, deg_hbm.at[pl.ds(cid * N + sid * SEG, tail)])


_deg_kernel = functools.partial(
    pl.kernel,
    out_type=jax.ShapeDtypeStruct((NC * N,), jnp.float32),
    mesh=_MESH,
    compiler_params=_SC_PARAMS,
    scratch_types=(
        [pltpu.VMEM((DEG_K,), jnp.float32),
         pltpu.VMEM((N - (NS - 1) * SEG,), jnp.float32),
         pltpu.VMEM((DEG_CH, DEG_K), jnp.int32),
         pltpu.VMEM_SHARED((N,), jnp.float32)]
        + [pltpu.SemaphoreType.DMA for _ in range(NBUF)]
    ),
)(_deg_body)


SCAT_K = 40
SCAT_EPT = E // NS
SCAT_CH = SCAT_EPT // SCAT_K
NSC = SCAT_CH // NBUF
LOOK = 2


def _scatter_body(W, g_hbm, src2_hbm, dst2_hbm, out_hbm, zbuf, raw, *ring):
    srcsc = ring[0:2]
    dstsc = ring[2:4]
    rows_v = ring[4:4 + NBUF]
    acc_sh = ring[4 + NBUF]
    gsem = ring[5 + NBUF:5 + 2 * NBUF]
    ssem = ring[5 + 2 * NBUF:5 + 3 * NBUF]
    cid = lax.axis_index("c")
    sid = lax.axis_index("s")
    off = cid * N

    _zero_buf(zbuf, ZRS, W)
    for k in range(RPT // ZRS):
        pltpu.sync_copy(zbuf, acc_sh.at[pl.ds(sid * RPT + k * ZRS, ZRS)])
    plsc.subcore_barrier()

    def load_sc(scj, p):
        base = sid * SCAT_CH + scj * NBUF
        pltpu.sync_copy(src2_hbm.at[pl.ds(base, NBUF)], raw)
        pltpu.sync_copy(dst2_hbm.at[pl.ds(base, NBUF)], dstsc[p])
        for r in range(NBUF):
            for o in (0, 16, 24):
                sl = pl.ds(o, LANES)
                srcsc[p][r, sl] = raw[r, sl] + off

    def issue_gather(set_, row, b):
        pltpu.async_copy(g_hbm.at[srcsc[set_].at[row]], rows_v[b], gsem[b])

    load_sc(0, 0)
    issue_gather(0, 0, 0)
    issue_gather(0, 1, 1)

    def body(j4, _):
        for par in range(2):
            j2 = j4 * 2 + par
            for b in range(NBUF):
                bi = (b + LOOK) % NBUF
                if b >= 3:
                    wset, wrow = par, b - 3
                else:
                    wset, wrow = 1 - par, b + 2

                @pl.when(j2 * NBUF + b - 3 >= 0)
                def _():
                    pltpu.make_async_copy(
                        rows_v[bi], acc_sh.at[dstsc[wset].at[wrow]],
                        ssem[bi]).wait()

                if b == 3:
                    @pl.when(j2 + 1 < NSC)
                    def _():
                        load_sc(j2 + 1, 1 - par)

                if b < 3:
                    iset, irow = par, b + LOOK
                else:
                    iset, irow = 1 - par, b - 3

                @pl.when(j2 * NBUF + b + LOOK < SCAT_CH)
                def _():
                    issue_gather(iset, irow, bi)

                pltpu.make_async_copy(
                    g_hbm.at[srcsc[par].at[b]], rows_v[b], gsem[b]).wait()
                pltpu.async_copy(
                    rows_v[b], acc_sh.at[dstsc[par].at[b]], ssem[b], add=True)
        return 0

    lax.fori_loop(0, NSC // 2, body, 0)
    for c in range(SCAT_CH - NBUF + LOOK, SCAT_CH):
        b = c % NBUF
        pltpu.make_async_copy(
            rows_v[b], acc_sh.at[dstsc[(c // NBUF) % 2].at[b]], ssem[b]).wait()
    plsc.subcore_barrier()

    for k in range(RPT // ZRS):
        r0 = sid * RPT + k * ZRS
        pltpu.sync_copy(acc_sh.at[pl.ds(r0, ZRS)], zbuf)
        pltpu.sync_copy(zbuf, out_hbm.at[pl.ds(off + r0, ZRS)])


def _make_scatter(W):
    return functools.partial(
        pl.kernel,
        out_type=jax.ShapeDtypeStruct((NC * N, W), jnp.float32),
        mesh=_MESH,
        compiler_params=_SC_PARAMS,
        scratch_types=(
            [pltpu.VMEM((ZRS, W), jnp.float32),
             pltpu.VMEM((NBUF, SCAT_K), jnp.int32)]
            + [pltpu.VMEM((NBUF, SCAT_K), jnp.int32) for _ in range(4)]
            + [pltpu.VMEM((SCAT_K, W), jnp.float32) for _ in range(NBUF)]
            + [pltpu.VMEM_SHARED((N, W), jnp.float32)]
            + [pltpu.SemaphoreType.DMA for _ in range(2 * NBUF)]
        ),
    )(functools.partial(_scatter_body, W))


_scatter128 = _make_scatter(H // NC)
_scatter32 = _make_scatter(C // NC)


def _dinv_block(deg_a, deg_b):
    return lax.rsqrt(deg_a + deg_b + 1.0)


def _mm1_body(x_ref, w_ref, dga_ref, dgb_ref, out_ref):
    dinv = _dinv_block(dga_ref[...], dgb_ref[...])
    h = jnp.dot(x_ref[...], w_ref[...], preferred_element_type=jnp.float32)
    out_ref[...] = dinv * h


_mm1 = pl.pallas_call(
    _mm1_body,
    grid=(NC, NB),
    in_specs=[
        pl.BlockSpec((BN, D), lambda p, i: (i, 0)),
        pl.BlockSpec((D, H // NC), lambda p, i: (0, p)),
        pl.BlockSpec((BN, 1), lambda p, i: (i, 0)),
        pl.BlockSpec((BN, 1), lambda p, i: (NB + i, 0)),
    ],
    out_specs=pl.BlockSpec((BN, H // NC), lambda p, i: (p * NB + i, 0)),
    out_shape=jax.ShapeDtypeStruct((NC * N, H // NC), jnp.float32),
)


def _mm2_body(s1a_ref, s1b_ref, g1a_ref, g1b_ref, w2t_ref, b1_ref,
              dga_ref, dgb_ref, out_ref):
    dinv = _dinv_block(dga_ref[...], dgb_ref[...])
    x1a = jax.nn.relu(dinv * (s1a_ref[...] + g1a_ref[...]) + b1_ref[0:1, 0:128])
    x1b = jax.nn.relu(dinv * (s1b_ref[...] + g1b_ref[...]) + b1_ref[0:1, 128:256])
    dn = (((1,), (1,)), ((), ()))
    acc = lax.dot_general(x1a, w2t_ref[:, 0:128], dn,
                          preferred_element_type=jnp.float32)
    acc += lax.dot_general(x1b, w2t_ref[:, 128:256], dn,
                           preferred_element_type=jnp.float32)
    out_ref[...] = dinv * acc


_mm2 = pl.pallas_call(
    _mm2_body,
    grid=(NC, NB),
    in_specs=[
        pl.BlockSpec((BN, H // NC), lambda p, i: (i, 0)),
        pl.BlockSpec((BN, H // NC), lambda p, i: (NB + i, 0)),
        pl.BlockSpec((BN, H // NC), lambda p, i: (i, 0)),
        pl.BlockSpec((BN, H // NC), lambda p, i: (NB + i, 0)),
        pl.BlockSpec((C // NC, H), lambda p, i: (p, 0)),
        pl.BlockSpec((1, H), lambda p, i: (0, 0)),
        pl.BlockSpec((BN, 1), lambda p, i: (i, 0)),
        pl.BlockSpec((BN, 1), lambda p, i: (NB + i, 0)),
    ],
    out_specs=pl.BlockSpec((BN, C // NC), lambda p, i: (p * NB + i, 0)),
    out_shape=jax.ShapeDtypeStruct((NC * N, C // NC), jnp.float32),
)


def _fin_body(s2a_ref, s2b_ref, g2a_ref, g2b_ref, b2_ref,
              dga_ref, dgb_ref, out_ref):
    dinv = _dinv_block(dga_ref[...], dgb_ref[...])
    ha = dinv * (s2a_ref[...] + g2a_ref[...]) + b2_ref[0:1, 0:32]
    hb = dinv * (s2b_ref[...] + g2b_ref[...]) + b2_ref[0:1, 32:64]
    out_ref[...] = jnp.concatenate([ha, hb], axis=1)


_fin = pl.pallas_call(
    _fin_body,
    grid=(NB,),
    in_specs=[
        pl.BlockSpec((BN, C // NC), lambda i: (i, 0)),
        pl.BlockSpec((BN, C // NC), lambda i: (NB + i, 0)),
        pl.BlockSpec((BN, C // NC), lambda i: (i, 0)),
        pl.BlockSpec((BN, C // NC), lambda i: (NB + i, 0)),
        pl.BlockSpec((1, C), lambda i: (0, 0)),
        pl.BlockSpec((BN, 1), lambda i: (i, 0)),
        pl.BlockSpec((BN, 1), lambda i: (NB + i, 0)),
    ],
    out_specs=pl.BlockSpec((BN, C), lambda i: (i, 0)),
    out_shape=jax.ShapeDtypeStruct((N, C), jnp.float32),
)


def kernel(last_e_emb, edge_index, W1, b1, W2, b2):
    src2 = edge_index[0].reshape(E // SCAT_K, SCAT_K)
    dst2 = edge_index[1].reshape(E // SCAT_K, SCAT_K)
    deg = _deg_kernel(dst2).reshape(NC * N, 1)
    g1 = _mm1(last_e_emb, W1, deg, deg)
    s1 = _scatter128(g1, src2, dst2)
    g2 = _mm2(s1, s1, g1, g1, W2.T, b1.reshape(1, H), deg, deg)
    s2 = _scatter32(g2, src2, dst2)
    return _fin(s2, s2, g2, g2, b2.reshape(1, C), deg, deg)


# speedup vs baseline: 17.2286x; 1.3090x over previous
import functools

import jax
import jax.numpy as jnp
from jax import lax
from jax.experimental import pallas as pl
from jax.experimental.pallas import tpu as pltpu
from jax.experimental.pallas import tpu_sc as plsc

N = 10000
E = 160000
D = 256
H = 256
C = 64

NC = 2
NS = 16
LANES = 16

BN = 2000
NB = N // BN
RPT = N // NS
ZRS = 125
_MESH = plsc.VectorSubcoreMesh(core_axis_name="c", subcore_axis_name="s")
_SC_PARAMS = pltpu.CompilerParams(use_tc_tiling_on_sc=False)


def _offs(k):
    offs = list(range(0, k - 15, 16))
    if offs[-1] + LANES < k:
        offs.append(k - LANES)
    return offs


def _zero_buf(buf, rows, width):
    zv = jnp.zeros((LANES,), jnp.float32)

    def body(r, _):
        for c in range(width // LANES):
            buf[r, pl.ds(c * LANES, LANES)] = zv
        return 0

    lax.fori_loop(0, rows, body, 0)


DEG_K = 50
DEG_EPT = E // (NC * NS)
DEG_CH = DEG_EPT // DEG_K

NBUF = 5


DEGW = 16


def _deg_body(edges_hbm, deg_hbm, ones_v, zbuf, idx2d, acc_sh, *ssem):
    cid = lax.axis_index("c")
    sid = lax.axis_index("s")
    wid = cid * NS + sid

    ov = jnp.ones((LANES,), jnp.float32)

    def fill_ones(r, _):
        ones_v[r, pl.ds(0, LANES)] = ov
        return 0

    lax.fori_loop(0, DEG_K, fill_ones, 0)
    _zero_buf(zbuf, RPT, DEGW)
    pltpu.sync_copy(zbuf, acc_sh.at[pl.ds(sid * RPT, RPT)])

    pltpu.sync_copy(edges_hbm.at[1, pl.ds(wid * DEG_CH, DEG_CH)], idx2d)
    plsc.subcore_barrier()

    def body(j2, _):
        for b in range(NBUF):
            j = j2 * NBUF + b

            @pl.when(j >= NBUF)
            def _():
                pltpu.make_async_copy(
                    ones_v, acc_sh.at[idx2d.at[j - NBUF]], ssem[b]).wait()

            pltpu.async_copy(ones_v, acc_sh.at[idx2d.at[j]], ssem[b], add=True)
        return 0

    lax.fori_loop(0, DEG_CH // NBUF, body, 0)
    for b in range(NBUF):
        pltpu.make_async_copy(ones_v, acc_sh.at[idx2d.at[0]], ssem[b]).wait()
    plsc.subcore_barrier()

    r0 = sid * RPT
    pltpu.sync_copy(acc_sh.at[pl.ds(r0, RPT)], zbuf)
    pltpu.sync_copy(zbuf, deg_hbm.at[pl.ds(cid * N + r0, RPT)])


_deg_kernel = functools.partial(
    pl.kernel,
    out_type=jax.ShapeDtypeStruct((NC * N, DEGW), jnp.float32),
    mesh=_MESH,
    compiler_params=_SC_PARAMS,
    scratch_types=(
        [pltpu.VMEM((DEG_K, DEGW), jnp.float32),
         pltpu.VMEM((RPT, DEGW), jnp.float32),
         pltpu.VMEM((DEG_CH, DEG_K), jnp.int32),
         pltpu.VMEM_SHARED((N, DEGW), jnp.float32)]
        + [pltpu.SemaphoreType.DMA for _ in range(NBUF)]
    ),
)(_deg_body)


SCAT_EPT = E // NS
LOOK = 2


def _scatter_body(W, K, g_hbm, edges_hbm, out_hbm, zbuf, raw, *ring):
    SCAT_CH = SCAT_EPT // K
    NSC = SCAT_CH // NBUF
    srcsc = ring[0:2]
    dstsc = ring[2:4]
    rows_v = ring[4:4 + NBUF]
    acc_sh = ring[4 + NBUF]
    gsem = ring[5 + NBUF:5 + 2 * NBUF]
    ssem = ring[5 + 2 * NBUF:5 + 3 * NBUF]
    cid = lax.axis_index("c")
    sid = lax.axis_index("s")
    off = cid * N

    _zero_buf(zbuf, ZRS, W)
    for k in range(RPT // ZRS):
        pltpu.sync_copy(zbuf, acc_sh.at[pl.ds(sid * RPT + k * ZRS, ZRS)])
    plsc.subcore_barrier()

    def load_sc(scj, p):
        base = sid * SCAT_CH + scj * NBUF
        pltpu.sync_copy(edges_hbm.at[0, pl.ds(base, NBUF)], raw)
        pltpu.sync_copy(edges_hbm.at[1, pl.ds(base, NBUF)], dstsc[p])
        for r in range(NBUF):
            for o in _offs(K):
                sl = pl.ds(o, LANES)
                srcsc[p][r, sl] = raw[r, sl] + off

    def issue_gather(set_, row, b):
        pltpu.async_copy(g_hbm.at[srcsc[set_].at[row]], rows_v[b], gsem[b])

    load_sc(0, 0)
    issue_gather(0, 0, 0)
    issue_gather(0, 1, 1)

    def body(j4, _):
        for par in range(2):
            j2 = j4 * 2 + par
            for b in range(NBUF):
                bi = (b + LOOK) % NBUF
                if b >= 3:
                    wset, wrow = par, b - 3
                else:
                    wset, wrow = 1 - par, b + 2

                @pl.when(j2 * NBUF + b - 3 >= 0)
                def _():
                    pltpu.make_async_copy(
                        rows_v[bi], acc_sh.at[dstsc[wset].at[wrow]],
                        ssem[bi]).wait()

                if b == 3:
                    @pl.when(j2 + 1 < NSC)
                    def _():
                        load_sc(j2 + 1, 1 - par)

                if b < 3:
                    iset, irow = par, b + LOOK
                else:
                    iset, irow = 1 - par, b - 3

                @pl.when(j2 * NBUF + b + LOOK < SCAT_CH)
                def _():
                    issue_gather(iset, irow, bi)

                pltpu.make_async_copy(
                    g_hbm.at[srcsc[par].at[b]], rows_v[b], gsem[b]).wait()
                pltpu.async_copy(
                    rows_v[b], acc_sh.at[dstsc[par].at[b]], ssem[b], add=True)
        return 0

    lax.fori_loop(0, NSC // 2, body, 0)
    for c in range(SCAT_CH - NBUF + LOOK, SCAT_CH):
        b = c % NBUF
        pltpu.make_async_copy(
            rows_v[b], acc_sh.at[dstsc[(c // NBUF) % 2].at[b]], ssem[b]).wait()
    plsc.subcore_barrier()

    for k in range(RPT // ZRS):
        r0 = sid * RPT + k * ZRS
        pltpu.sync_copy(acc_sh.at[pl.ds(r0, ZRS)], zbuf)
        pltpu.sync_copy(zbuf, out_hbm.at[pl.ds(off + r0, ZRS)])


def _make_scatter(W, K):
    return functools.partial(
        pl.kernel,
        out_type=jax.ShapeDtypeStruct((NC * N, W), jnp.float32),
        mesh=_MESH,
        compiler_params=_SC_PARAMS,
        scratch_types=(
            [pltpu.VMEM((ZRS, W), jnp.float32),
             pltpu.VMEM((NBUF, K), jnp.int32)]
            + [pltpu.VMEM((NBUF, K), jnp.int32) for _ in range(4)]
            + [pltpu.VMEM((K, W), jnp.float32) for _ in range(NBUF)]
            + [pltpu.VMEM_SHARED((N, W), jnp.float32)]
            + [pltpu.SemaphoreType.DMA for _ in range(2 * NBUF)]
        ),
    )(functools.partial(_scatter_body, W, K))


K1 = 50
K2 = 100
_scatter128 = _make_scatter(H // NC, K1)
_scatter32 = _make_scatter(C // NC, K2)


def _dinv_block(deg_a, deg_b):
    return lax.rsqrt(deg_a[:, 0:1] + deg_b[:, 0:1] + 1.0)


def _mm1_body(x_ref, w_ref, dga_ref, dgb_ref, out_ref):
    dinv = _dinv_block(dga_ref[...], dgb_ref[...])
    h = jnp.dot(x_ref[...], w_ref[...], preferred_element_type=jnp.float32)
    out_ref[...] = dinv * h


_mm1 = pl.pallas_call(
    _mm1_body,
    grid=(NC, NB),
    in_specs=[
        pl.BlockSpec((BN, D), lambda p, i: (i, 0)),
        pl.BlockSpec((D, H // NC), lambda p, i: (0, p)),
        pl.BlockSpec((BN, DEGW), lambda p, i: (i, 0)),
        pl.BlockSpec((BN, DEGW), lambda p, i: (NB + i, 0)),
    ],
    out_specs=pl.BlockSpec((BN, H // NC), lambda p, i: (p * NB + i, 0)),
    out_shape=jax.ShapeDtypeStruct((NC * N, H // NC), jnp.float32),
)


def _mm2_body(s1a_ref, s1b_ref, g1a_ref, g1b_ref, w2t_ref, b1_ref,
              dga_ref, dgb_ref, out_ref):
    dinv = _dinv_block(dga_ref[...], dgb_ref[...])
    x1a = jax.nn.relu(dinv * (s1a_ref[...] + g1a_ref[...]) + b1_ref[0:1, 0:128])
    x1b = jax.nn.relu(dinv * (s1b_ref[...] + g1b_ref[...]) + b1_ref[0:1, 128:256])
    dn = (((1,), (1,)), ((), ()))
    acc = lax.dot_general(x1a, w2t_ref[:, 0:128], dn,
                          preferred_element_type=jnp.float32)
    acc += lax.dot_general(x1b, w2t_ref[:, 128:256], dn,
                           preferred_element_type=jnp.float32)
    out_ref[...] = dinv * acc


_mm2 = pl.pallas_call(
    _mm2_body,
    grid=(NC, NB),
    in_specs=[
        pl.BlockSpec((BN, H // NC), lambda p, i: (i, 0)),
        pl.BlockSpec((BN, H // NC), lambda p, i: (NB + i, 0)),
        pl.BlockSpec((BN, H // NC), lambda p, i: (i, 0)),
        pl.BlockSpec((BN, H // NC), lambda p, i: (NB + i, 0)),
        pl.BlockSpec((C // NC, H), lambda p, i: (p, 0)),
        pl.BlockSpec((1, H), lambda p, i: (0, 0)),
        pl.BlockSpec((BN, DEGW), lambda p, i: (i, 0)),
        pl.BlockSpec((BN, DEGW), lambda p, i: (NB + i, 0)),
    ],
    out_specs=pl.BlockSpec((BN, C // NC), lambda p, i: (p * NB + i, 0)),
    out_shape=jax.ShapeDtypeStruct((NC * N, C // NC), jnp.float32),
)


def _fin_body(s2a_ref, s2b_ref, g2a_ref, g2b_ref, b2_ref,
              dga_ref, dgb_ref, out_ref):
    dinv = _dinv_block(dga_ref[...], dgb_ref[...])
    ha = dinv * (s2a_ref[...] + g2a_ref[...]) + b2_ref[0:1, 0:32]
    hb = dinv * (s2b_ref[...] + g2b_ref[...]) + b2_ref[0:1, 32:64]
    out_ref[...] = jnp.concatenate([ha, hb], axis=1)


_fin = pl.pallas_call(
    _fin_body,
    grid=(NB,),
    in_specs=[
        pl.BlockSpec((BN, C // NC), lambda i: (i, 0)),
        pl.BlockSpec((BN, C // NC), lambda i: (NB + i, 0)),
        pl.BlockSpec((BN, C // NC), lambda i: (i, 0)),
        pl.BlockSpec((BN, C // NC), lambda i: (NB + i, 0)),
        pl.BlockSpec((1, C), lambda i: (0, 0)),
        pl.BlockSpec((BN, DEGW), lambda i: (i, 0)),
        pl.BlockSpec((BN, DEGW), lambda i: (NB + i, 0)),
    ],
    out_specs=pl.BlockSpec((BN, C), lambda i: (i, 0)),
    out_shape=jax.ShapeDtypeStruct((N, C), jnp.float32),
)


def kernel(last_e_emb, edge_index, W1, b1, W2, b2):
    edges3a = edge_index.reshape(2, E // K1, K1)
    edges3b = edge_index.reshape(2, E // K2, K2)
    deg = _deg_kernel(edges3a)
    g1 = _mm1(last_e_emb, W1, deg, deg)
    s1 = _scatter128(g1, edges3a)
    g2 = _mm2(s1, s1, g1, g1, W2.T, b1.reshape(1, H), deg, deg)
    s2 = _scatter32(g2, edges3b)
    return _fin(s2, s2, g2, g2, b2.reshape(1, C), deg, deg)


# speedup vs baseline: 18.2488x; 1.0592x over previous
import functools

import jax
import jax.numpy as jnp
from jax import lax
from jax.experimental import pallas as pl
from jax.experimental.pallas import tpu as pltpu
from jax.experimental.pallas import tpu_sc as plsc

N = 10000
E = 160000
D = 256
H = 256
C = 64

NC = 2
NS = 16
LANES = 16

BN = 2000
NB = N // BN
RPT = N // NS
ZRS = 125
SEG = 624

_MESH = plsc.VectorSubcoreMesh(core_axis_name="c", subcore_axis_name="s")
_SC_PARAMS = pltpu.CompilerParams(use_tc_tiling_on_sc=False)


def _offs(k):
    offs = list(range(0, k - 15, 16))
    if offs[-1] + LANES < k:
        offs.append(k - LANES)
    return offs


def _zero_buf(buf, rows, width):
    zv = jnp.zeros((LANES,), jnp.float32)

    def body(r, _):
        for c in range(width // LANES):
            buf[r, pl.ds(c * LANES, LANES)] = zv
        return 0

    lax.fori_loop(0, rows, body, 0)


DEG_K = 50
DEG_EPT = E // (NC * NS)
DEG_CH = DEG_EPT // DEG_K

NBUF = 5


def _deg_body(edges_hbm, deg_hbm, ones_v, zbuf, idx2d, acc_sh, *ssem):
    cid = lax.axis_index("c")
    sid = lax.axis_index("s")
    wid = cid * NS + sid

    ov = jnp.ones((LANES,), jnp.float32)
    for o in _offs(DEG_K):
        ones_v[pl.ds(o, LANES)] = ov
    zv = jnp.zeros((LANES,), jnp.float32)
    for o in range(0, SEG + LANES, LANES):
        zbuf[pl.ds(o, LANES)] = zv

    @pl.when(sid < NS - 1)
    def _():
        pltpu.sync_copy(zbuf.at[pl.ds(0, SEG)], acc_sh.at[pl.ds(sid * SEG, SEG)])

    @pl.when(sid == NS - 1)
    def _():
        pltpu.sync_copy(zbuf, acc_sh.at[pl.ds(sid * SEG, N - (NS - 1) * SEG)])

    pltpu.sync_copy(edges_hbm.at[1, pl.ds(wid * DEG_CH, DEG_CH)], idx2d)
    plsc.subcore_barrier()

    def body(j2, _):
        for b in range(NBUF):
            j = j2 * NBUF + b

            @pl.when(j >= NBUF)
            def _():
                pltpu.make_async_copy(
                    ones_v, acc_sh.at[idx2d.at[j - NBUF]], ssem[b]).wait()

            pltpu.async_copy(ones_v, acc_sh.at[idx2d.at[j]], ssem[b], add=True)
        return 0

    lax.fori_loop(0, DEG_CH // NBUF, body, 0)
    for b in range(NBUF):
        pltpu.make_async_copy(ones_v, acc_sh.at[idx2d.at[0]], ssem[b]).wait()
    plsc.subcore_barrier()

    @pl.when(sid < NS - 1)
    def _():
        pltpu.sync_copy(acc_sh.at[pl.ds(sid * SEG, SEG)], zbuf.at[pl.ds(0, SEG)])
        pltpu.sync_copy(zbuf.at[pl.ds(0, SEG)],
                        deg_hbm.at[pl.ds(cid * N + sid * SEG, SEG)])

    @pl.when(sid == NS - 1)
    def _():
        tail = N - (NS - 1) * SEG
        pltpu.sync_copy(acc_sh.at[pl.ds(sid * SEG, tail)], zbuf)
        pltpu.sync_copy(zbuf, deg_hbm.at[pl.ds(cid * N + sid * SEG, tail)])


_deg_kernel = functools.partial(
    pl.kernel,
    out_type=jax.ShapeDtypeStruct((NC * N,), jnp.float32),
    mesh=_MESH,
    compiler_params=_SC_PARAMS,
    scratch_types=(
        [pltpu.VMEM((DEG_K,), jnp.float32),
         pltpu.VMEM((N - (NS - 1) * SEG,), jnp.float32),
         pltpu.VMEM((DEG_CH, DEG_K), jnp.int32),
         pltpu.VMEM_SHARED((N,), jnp.float32)]
        + [pltpu.SemaphoreType.DMA for _ in range(NBUF)]
    ),
)(_deg_body)


SCAT_EPT = E // NS
LOOK = 2


def _scatter_body(W, K, g_hbm, edges_hbm, out_hbm, zbuf, raw, *ring):
    SCAT_CH = SCAT_EPT // K
    NSC = SCAT_CH // NBUF
    srcsc = ring[0:2]
    dstsc = ring[2:4]
    rows_v = ring[4:4 + NBUF]
    acc_sh = ring[4 + NBUF]
    gsem = ring[5 + NBUF:5 + 2 * NBUF]
    ssem = ring[5 + 2 * NBUF:5 + 3 * NBUF]
    cid = lax.axis_index("c")
    sid = lax.axis_index("s")
    off = cid * N

    _zero_buf(zbuf, ZRS, W)
    for k in range(RPT // ZRS):
        pltpu.sync_copy(zbuf, acc_sh.at[pl.ds(sid * RPT + k * ZRS, ZRS)])
    plsc.subcore_barrier()

    def load_sc(scj, p):
        base = sid * SCAT_CH + scj * NBUF
        pltpu.sync_copy(edges_hbm.at[0, pl.ds(base, NBUF)], raw)
        pltpu.sync_copy(edges_hbm.at[1, pl.ds(base, NBUF)], dstsc[p])
        for r in range(NBUF):
            for o in _offs(K):
                sl = pl.ds(o, LANES)
                srcsc[p][r, sl] = raw[r, sl] + off

    def issue_gather(set_, row, b):
        pltpu.async_copy(g_hbm.at[srcsc[set_].at[row]], rows_v[b], gsem[b])

    load_sc(0, 0)
    issue_gather(0, 0, 0)
    issue_gather(0, 1, 1)

    def body(j4, _):
        for par in range(2):
            j2 = j4 * 2 + par
            for b in range(NBUF):
                bi = (b + LOOK) % NBUF
                if b >= 3:
                    wset, wrow = par, b - 3
                else:
                    wset, wrow = 1 - par, b + 2

                @pl.when(j2 * NBUF + b - 3 >= 0)
                def _():
                    pltpu.make_async_copy(
                        rows_v[bi], acc_sh.at[dstsc[wset].at[wrow]],
                        ssem[bi]).wait()

                if b == 3:
                    @pl.when(j2 + 1 < NSC)
                    def _():
                        load_sc(j2 + 1, 1 - par)

                if b < 3:
                    iset, irow = par, b + LOOK
                else:
                    iset, irow = 1 - par, b - 3

                @pl.when(j2 * NBUF + b + LOOK < SCAT_CH)
                def _():
                    issue_gather(iset, irow, bi)

                pltpu.make_async_copy(
                    g_hbm.at[srcsc[par].at[b]], rows_v[b], gsem[b]).wait()
                pltpu.async_copy(
                    rows_v[b], acc_sh.at[dstsc[par].at[b]], ssem[b], add=True)
        return 0

    lax.fori_loop(0, NSC // 2, body, 0)
    for c in range(SCAT_CH - NBUF + LOOK, SCAT_CH):
        b = c % NBUF
        pltpu.make_async_copy(
            rows_v[b], acc_sh.at[dstsc[(c // NBUF) % 2].at[b]], ssem[b]).wait()
    plsc.subcore_barrier()

    for k in range(RPT // ZRS):
        r0 = sid * RPT + k * ZRS
        pltpu.sync_copy(acc_sh.at[pl.ds(r0, ZRS)], zbuf)
        pltpu.sync_copy(zbuf, out_hbm.at[pl.ds(off + r0, ZRS)])


def _make_scatter(W, K):
    return functools.partial(
        pl.kernel,
        out_type=jax.ShapeDtypeStruct((NC * N, W), jnp.float32),
        mesh=_MESH,
        compiler_params=_SC_PARAMS,
        scratch_types=(
            [pltpu.VMEM((ZRS, W), jnp.float32),
             pltpu.VMEM((NBUF, K), jnp.int32)]
            + [pltpu.VMEM((NBUF, K), jnp.int32) for _ in range(4)]
            + [pltpu.VMEM((K, W), jnp.float32) for _ in range(NBUF)]
            + [pltpu.VMEM_SHARED((N, W), jnp.float32)]
            + [pltpu.SemaphoreType.DMA for _ in range(2 * NBUF)]
        ),
    )(functools.partial(_scatter_body, W, K))


K1 = 50
K2 = 100
_scatter128 = _make_scatter(H // NC, K1)
_scatter32 = _make_scatter(C // NC, K2)


def _dinv_block(deg_a, deg_b):
    return lax.rsqrt(deg_a + deg_b + 1.0)


def _mm1_body(x_ref, w_ref, dg_ref, out_ref):
    dinv = lax.rsqrt(dg_ref[0] + dg_ref[1] + 1.0)
    h = jnp.dot(x_ref[...], w_ref[...], preferred_element_type=jnp.float32)
    out_ref[0] = dinv * h[:, 0:128]
    out_ref[1] = dinv * h[:, 128:256]


_mm1 = pl.pallas_call(
    _mm1_body,
    grid=(NB,),
    in_specs=[
        pl.BlockSpec((BN, D), lambda i: (i, 0)),
        pl.BlockSpec((D, H), lambda i: (0, 0)),
        pl.BlockSpec((2, BN, 1), lambda i: (0, i, 0)),
    ],
    out_specs=pl.BlockSpec((2, BN, H // NC), lambda i: (0, i, 0)),
    out_shape=jax.ShapeDtypeStruct((NC, N, H // NC), jnp.float32),
)


def _mm2_body(s1_ref, g1_ref, w2_ref, b1_ref, dg_ref, out_ref):
    dinv = lax.rsqrt(dg_ref[0] + dg_ref[1] + 1.0)
    x1a = jax.nn.relu(dinv * (s1_ref[0] + g1_ref[0]) + b1_ref[0:1, 0:128])
    x1b = jax.nn.relu(dinv * (s1_ref[1] + g1_ref[1]) + b1_ref[0:1, 128:256])
    x1 = jnp.concatenate([x1a, x1b], axis=1)
    acc = jnp.dot(x1, w2_ref[...], preferred_element_type=jnp.float32)
    out_ref[0] = dinv * acc[:, 0:32]
    out_ref[1] = dinv * acc[:, 32:64]


_mm2 = pl.pallas_call(
    _mm2_body,
    grid=(NB,),
    in_specs=[
        pl.BlockSpec((2, BN, H // NC), lambda i: (0, i, 0)),
        pl.BlockSpec((2, BN, H // NC), lambda i: (0, i, 0)),
        pl.BlockSpec((H, C), lambda i: (0, 0)),
        pl.BlockSpec((1, H), lambda i: (0, 0)),
        pl.BlockSpec((2, BN, 1), lambda i: (0, i, 0)),
    ],
    out_specs=pl.BlockSpec((2, BN, C // NC), lambda i: (0, i, 0)),
    out_shape=jax.ShapeDtypeStruct((NC, N, C // NC), jnp.float32),
)


def _fin_body(s2_ref, g2_ref, b2_ref, dg_ref, out_ref):
    dinv = lax.rsqrt(dg_ref[0] + dg_ref[1] + 1.0)
    ha = dinv * (s2_ref[0] + g2_ref[0]) + b2_ref[0:1, 0:32]
    hb = dinv * (s2_ref[1] + g2_ref[1]) + b2_ref[0:1, 32:64]
    out_ref[...] = jnp.concatenate([ha, hb], axis=1)


_fin = pl.pallas_call(
    _fin_body,
    grid=(NB,),
    in_specs=[
        pl.BlockSpec((2, BN, C // NC), lambda i: (0, i, 0)),
        pl.BlockSpec((2, BN, C // NC), lambda i: (0, i, 0)),
        pl.BlockSpec((1, C), lambda i: (0, 0)),
        pl.BlockSpec((2, BN, 1), lambda i: (0, i, 0)),
    ],
    out_specs=pl.BlockSpec((BN, C), lambda i: (i, 0)),
    out_shape=jax.ShapeDtypeStruct((N, C), jnp.float32),
)


def kernel(last_e_emb, edge_index, W1, b1, W2, b2):
    edges3a = edge_index.reshape(2, E // K1, K1)
    edges3b = edge_index.reshape(2, E // K2, K2)
    deg = _deg_kernel(edges3a).reshape(NC, N, 1)
    g1 = _mm1(last_e_emb, W1, deg)
    s1 = _scatter128(g1.reshape(NC * N, H // NC), edges3a)
    s1v = s1.reshape(NC, N, H // NC)
    g2 = _mm2(s1v, g1, W2, b1.reshape(1, H), deg)
    s2 = _scatter32(g2.reshape(NC * N, C // NC), edges3b)
    return _fin(s2.reshape(NC, N, C // NC), g2, b2.reshape(1, C), deg)


# speedup vs baseline: 19.0020x; 1.0413x over previous
import functools

import jax
import jax.numpy as jnp
from jax import lax
from jax.experimental import pallas as pl
from jax.experimental.pallas import tpu as pltpu
from jax.experimental.pallas import tpu_sc as plsc

N = 10000
E = 160000
D = 256
H = 256
C = 64

NC = 2
NS = 16
LANES = 16

BN = 2000
NB = N // BN
RPT = N // NS
ZRS = 125
SEG = 624

_MESH = plsc.VectorSubcoreMesh(core_axis_name="c", subcore_axis_name="s")
_SC_PARAMS = pltpu.CompilerParams(use_tc_tiling_on_sc=False)


def _offs(k):
    offs = list(range(0, k - 15, 16))
    if offs[-1] + LANES < k:
        offs.append(k - LANES)
    return offs


def _zero_buf(buf, rows, width):
    zv = jnp.zeros((LANES,), jnp.float32)

    def body(r, _):
        for c in range(width // LANES):
            buf[r, pl.ds(c * LANES, LANES)] = zv
        return 0

    lax.fori_loop(0, rows, body, 0)


DEG_K = 50
DEG_EPT = E // (NC * NS)
DEG_CH = DEG_EPT // DEG_K

NBUF = 5


def _deg_body(edges_hbm, deg_hbm, ones_v, zbuf, idx2d, acc_sh, *ssem):
    cid = lax.axis_index("c")
    sid = lax.axis_index("s")
    wid = cid * NS + sid

    ov = jnp.ones((LANES,), jnp.float32)
    for o in _offs(DEG_K):
        ones_v[pl.ds(o, LANES)] = ov
    zv = jnp.zeros((LANES,), jnp.float32)
    for o in range(0, SEG + LANES, LANES):
        zbuf[pl.ds(o, LANES)] = zv

    @pl.when(sid < NS - 1)
    def _():
        pltpu.sync_copy(zbuf.at[pl.ds(0, SEG)], acc_sh.at[pl.ds(sid * SEG, SEG)])

    @pl.when(sid == NS - 1)
    def _():
        pltpu.sync_copy(zbuf, acc_sh.at[pl.ds(sid * SEG, N - (NS - 1) * SEG)])

    pltpu.sync_copy(edges_hbm.at[1, pl.ds(wid * DEG_CH, DEG_CH)], idx2d)
    plsc.subcore_barrier()

    def body(j2, _):
        for b in range(NBUF):
            j = j2 * NBUF + b

            @pl.when(j >= NBUF)
            def _():
                pltpu.make_async_copy(
                    ones_v, acc_sh.at[idx2d.at[j - NBUF]], ssem[b]).wait()

            pltpu.async_copy(ones_v, acc_sh.at[idx2d.at[j]], ssem[b], add=True)
        return 0

    lax.fori_loop(0, DEG_CH // NBUF, body, 0)
    for b in range(NBUF):
        pltpu.make_async_copy(ones_v, acc_sh.at[idx2d.at[0]], ssem[b]).wait()
    plsc.subcore_barrier()

    @pl.when(sid < NS - 1)
    def _():
        pltpu.sync_copy(acc_sh.at[pl.ds(sid * SEG, SEG)], zbuf.at[pl.ds(0, SEG)])
        pltpu.sync_copy(zbuf.at[pl.ds(0, SEG)],
                        deg_hbm.at[pl.ds(cid * N + sid * SEG, SEG)])

    @pl.when(sid == NS - 1)
    def _():
        tail = N - (NS - 1) * SEG
        pltpu.sync_copy(acc_sh.at[pl.ds(sid * SEG, tail)], zbuf)
        pltpu.sync_copy(zbuf, deg_hbm.at[pl.ds(cid * N + sid * SEG, tail)])


_deg_kernel = functools.partial(
    pl.kernel,
    out_type=jax.ShapeDtypeStruct((NC * N,), jnp.float32),
    mesh=_MESH,
    compiler_params=_SC_PARAMS,
    scratch_types=(
        [pltpu.VMEM((DEG_K,), jnp.float32),
         pltpu.VMEM((N - (NS - 1) * SEG,), jnp.float32),
         pltpu.VMEM((DEG_CH, DEG_K), jnp.int32),
         pltpu.VMEM_SHARED((N,), jnp.float32)]
        + [pltpu.SemaphoreType.DMA for _ in range(NBUF)]
    ),
)(_deg_body)


SCAT_EPT = E // NS
LOOK = 3


def _scatter_body(W, K, g_hbm, edges_hbm, out_hbm, zbuf, raw, *ring):
    SCAT_CH = SCAT_EPT // K
    NSC = SCAT_CH // NBUF
    srcsc = ring[0:2]
    dstsc = ring[2:4]
    rows_v = ring[4:4 + NBUF]
    acc_sh = ring[4 + NBUF]
    gsem = ring[5 + NBUF:5 + 2 * NBUF]
    ssem = ring[5 + 2 * NBUF:5 + 3 * NBUF]
    cid = lax.axis_index("c")
    sid = lax.axis_index("s")
    off = cid * N

    _zero_buf(zbuf, ZRS, W)
    for k in range(RPT // ZRS):
        pltpu.sync_copy(zbuf, acc_sh.at[pl.ds(sid * RPT + k * ZRS, ZRS)])
    plsc.subcore_barrier()

    def load_sc(scj, p):
        base = sid * SCAT_CH + scj * NBUF
        pltpu.sync_copy(edges_hbm.at[0, pl.ds(base, NBUF)], raw)
        pltpu.sync_copy(edges_hbm.at[1, pl.ds(base, NBUF)], dstsc[p])
        for r in range(NBUF):
            for o in _offs(K):
                sl = pl.ds(o, LANES)
                srcsc[p][r, sl] = raw[r, sl] + off

    def issue_gather(set_, row, b):
        pltpu.async_copy(g_hbm.at[srcsc[set_].at[row]], rows_v[b], gsem[b])

    WT = NBUF - LOOK

    load_sc(0, 0)
    for c0 in range(LOOK):
        issue_gather(0, c0, c0)

    def body(j4, _):
        for par in range(2):
            j2 = j4 * 2 + par
            for b in range(NBUF):
                bi = (b + LOOK) % NBUF
                if b >= WT:
                    wset, wrow = par, b - WT
                else:
                    wset, wrow = 1 - par, b + LOOK

                @pl.when(j2 * NBUF + b - WT >= 0)
                def _():
                    pltpu.make_async_copy(
                        rows_v[bi], acc_sh.at[dstsc[wset].at[wrow]],
                        ssem[bi]).wait()

                if b == WT:
                    @pl.when(j2 + 1 < NSC)
                    def _():
                        load_sc(j2 + 1, 1 - par)

                if b < WT:
                    iset, irow = par, b + LOOK
                else:
                    iset, irow = 1 - par, b - WT

                @pl.when(j2 * NBUF + b + LOOK < SCAT_CH)
                def _():
                    issue_gather(iset, irow, bi)

                pltpu.make_async_copy(
                    g_hbm.at[srcsc[par].at[b]], rows_v[b], gsem[b]).wait()
                pltpu.async_copy(
                    rows_v[b], acc_sh.at[dstsc[par].at[b]], ssem[b], add=True)
        return 0

    lax.fori_loop(0, NSC // 2, body, 0)
    for c in range(SCAT_CH - NBUF + LOOK, SCAT_CH):
        b = c % NBUF
        pltpu.make_async_copy(
            rows_v[b], acc_sh.at[dstsc[(c // NBUF) % 2].at[b]], ssem[b]).wait()
    plsc.subcore_barrier()

    for k in range(RPT // ZRS):
        r0 = sid * RPT + k * ZRS
        pltpu.sync_copy(acc_sh.at[pl.ds(r0, ZRS)], zbuf)
        pltpu.sync_copy(zbuf, out_hbm.at[pl.ds(off + r0, ZRS)])


def _make_scatter(W, K):
    return functools.partial(
        pl.kernel,
        out_type=jax.ShapeDtypeStruct((NC * N, W), jnp.float32),
        mesh=_MESH,
        compiler_params=_SC_PARAMS,
        scratch_types=(
            [pltpu.VMEM((ZRS, W), jnp.float32),
             pltpu.VMEM((NBUF, K), jnp.int32)]
            + [pltpu.VMEM((NBUF, K), jnp.int32) for _ in range(4)]
            + [pltpu.VMEM((K, W), jnp.float32) for _ in range(NBUF)]
            + [pltpu.VMEM_SHARED((N, W), jnp.float32)]
            + [pltpu.SemaphoreType.DMA for _ in range(2 * NBUF)]
        ),
    )(functools.partial(_scatter_body, W, K))


K1 = 50
K2 = 100
_scatter128 = _make_scatter(H // NC, K1)
_scatter32 = _make_scatter(C // NC, K2)


def _dinv_block(deg_a, deg_b):
    return lax.rsqrt(deg_a + deg_b + 1.0)


def _mm1_body(x_ref, w_ref, dg_ref, out_ref):
    dinv = lax.rsqrt(dg_ref[0] + dg_ref[1] + 1.0)
    h = jnp.dot(x_ref[...], w_ref[...], preferred_element_type=jnp.float32)
    out_ref[0] = dinv * h[:, 0:128]
    out_ref[1] = dinv * h[:, 128:256]


_mm1 = pl.pallas_call(
    _mm1_body,
    grid=(NB,),
    in_specs=[
        pl.BlockSpec((BN, D), lambda i: (i, 0)),
        pl.BlockSpec((D, H), lambda i: (0, 0)),
        pl.BlockSpec((2, BN, 1), lambda i: (0, i, 0)),
    ],
    out_specs=pl.BlockSpec((2, BN, H // NC), lambda i: (0, i, 0)),
    out_shape=jax.ShapeDtypeStruct((NC, N, H // NC), jnp.float32),
)


def _mm2_body(s1_ref, g1_ref, w2_ref, b1_ref, dg_ref, out_ref):
    dinv = lax.rsqrt(dg_ref[0] + dg_ref[1] + 1.0)
    x1a = jax.nn.relu(dinv * (s1_ref[0] + g1_ref[0]) + b1_ref[0:1, 0:128])
    x1b = jax.nn.relu(dinv * (s1_ref[1] + g1_ref[1]) + b1_ref[0:1, 128:256])
    x1 = jnp.concatenate([x1a, x1b], axis=1)
    acc = jnp.dot(x1, w2_ref[...], preferred_element_type=jnp.float32)
    out_ref[0] = dinv * acc[:, 0:32]
    out_ref[1] = dinv * acc[:, 32:64]


_mm2 = pl.pallas_call(
    _mm2_body,
    grid=(NB,),
    in_specs=[
        pl.BlockSpec((2, BN, H // NC), lambda i: (0, i, 0)),
        pl.BlockSpec((2, BN, H // NC), lambda i: (0, i, 0)),
        pl.BlockSpec((H, C), lambda i: (0, 0)),
        pl.BlockSpec((1, H), lambda i: (0, 0)),
        pl.BlockSpec((2, BN, 1), lambda i: (0, i, 0)),
    ],
    out_specs=pl.BlockSpec((2, BN, C // NC), lambda i: (0, i, 0)),
    out_shape=jax.ShapeDtypeStruct((NC, N, C // NC), jnp.float32),
)


def _fin_body(s2_ref, g2_ref, b2_ref, dg_ref, out_ref):
    dinv = lax.rsqrt(dg_ref[0] + dg_ref[1] + 1.0)
    ha = dinv * (s2_ref[0] + g2_ref[0]) + b2_ref[0:1, 0:32]
    hb = dinv * (s2_ref[1] + g2_ref[1]) + b2_ref[0:1, 32:64]
    out_ref[...] = jnp.concatenate([ha, hb], axis=1)


_fin = pl.pallas_call(
    _fin_body,
    grid=(NB,),
    in_specs=[
        pl.BlockSpec((2, BN, C // NC), lambda i: (0, i, 0)),
        pl.BlockSpec((2, BN, C // NC), lambda i: (0, i, 0)),
        pl.BlockSpec((1, C), lambda i: (0, 0)),
        pl.BlockSpec((2, BN, 1), lambda i: (0, i, 0)),
    ],
    out_specs=pl.BlockSpec((BN, C), lambda i: (i, 0)),
    out_shape=jax.ShapeDtypeStruct((N, C), jnp.float32),
)


def kernel(last_e_emb, edge_index, W1, b1, W2, b2):
    edges3a = edge_index.reshape(2, E // K1, K1)
    edges3b = edge_index.reshape(2, E // K2, K2)
    deg = _deg_kernel(edges3a).reshape(NC, N, 1)
    g1 = _mm1(last_e_emb, W1, deg)
    s1 = _scatter128(g1.reshape(NC * N, H // NC), edges3a)
    s1v = s1.reshape(NC, N, H // NC)
    g2 = _mm2(s1v, g1, W2, b1.reshape(1, H), deg)
    s2 = _scatter32(g2.reshape(NC * N, C // NC), edges3b)
    return _fin(s2.reshape(NC, N, C // NC), g2, b2.reshape(1, C), deg)


# speedup vs baseline: 19.0026x; 1.0000x over previous
import functools

import jax
import jax.numpy as jnp
from jax import lax
from jax.experimental import pallas as pl
from jax.experimental.pallas import tpu as pltpu
from jax.experimental.pallas import tpu_sc as plsc

N = 10000
E = 160000
D = 256
H = 256
C = 64

NC = 2
NS = 16
LANES = 16

BN = 2000
NB = N // BN
RPT = N // NS
ZRS = 125
SEG = 624

_MESH = plsc.VectorSubcoreMesh(core_axis_name="c", subcore_axis_name="s")
_SC_PARAMS = pltpu.CompilerParams(use_tc_tiling_on_sc=False)


def _offs(k):
    offs = list(range(0, k - 15, 16))
    if offs[-1] + LANES < k:
        offs.append(k - LANES)
    return offs


def _zero_buf(buf, rows, width):
    zv = jnp.zeros((LANES,), jnp.float32)

    def body(r, _):
        for c in range(width // LANES):
            buf[r, pl.ds(c * LANES, LANES)] = zv
        return 0

    lax.fori_loop(0, rows, body, 0)


DEG_K = 50
DEG_EPT = E // (NC * NS)
DEG_CH = DEG_EPT // DEG_K

NBUF = 5


def _deg_body(edges_hbm, deg_hbm, ones_v, zbuf, idx2d, acc_sh, *ssem):
    cid = lax.axis_index("c")
    sid = lax.axis_index("s")
    wid = cid * NS + sid

    ov = jnp.ones((LANES,), jnp.float32)
    for o in _offs(DEG_K):
        ones_v[pl.ds(o, LANES)] = ov
    zv = jnp.zeros((LANES,), jnp.float32)
    for o in range(0, SEG + LANES, LANES):
        zbuf[pl.ds(o, LANES)] = zv

    @pl.when(sid < NS - 1)
    def _():
        pltpu.sync_copy(zbuf.at[pl.ds(0, SEG)], acc_sh.at[pl.ds(sid * SEG, SEG)])

    @pl.when(sid == NS - 1)
    def _():
        pltpu.sync_copy(zbuf, acc_sh.at[pl.ds(sid * SEG, N - (NS - 1) * SEG)])

    pltpu.sync_copy(edges_hbm.at[1, pl.ds(wid * DEG_CH, DEG_CH)], idx2d)
    plsc.subcore_barrier()

    def body(j2, _):
        for b in range(NBUF):
            j = j2 * NBUF + b

            @pl.when(j >= NBUF)
            def _():
                pltpu.make_async_copy(
                    ones_v, acc_sh.at[idx2d.at[j - NBUF]], ssem[b]).wait()

            pltpu.async_copy(ones_v, acc_sh.at[idx2d.at[j]], ssem[b], add=True)
        return 0

    lax.fori_loop(0, DEG_CH // NBUF, body, 0)
    for b in range(NBUF):
        pltpu.make_async_copy(ones_v, acc_sh.at[idx2d.at[0]], ssem[b]).wait()
    plsc.subcore_barrier()

    @pl.when(sid < NS - 1)
    def _():
        pltpu.sync_copy(acc_sh.at[pl.ds(sid * SEG, SEG)], zbuf.at[pl.ds(0, SEG)])
        pltpu.sync_copy(zbuf.at[pl.ds(0, SEG)],
                        deg_hbm.at[pl.ds(cid * N + sid * SEG, SEG)])

    @pl.when(sid == NS - 1)
    def _():
        tail = N - (NS - 1) * SEG
        pltpu.sync_copy(acc_sh.at[pl.ds(sid * SEG, tail)], zbuf)
        pltpu.sync_copy(zbuf, deg_hbm.at[pl.ds(cid * N + sid * SEG, tail)])


_deg_kernel = functools.partial(
    pl.kernel,
    out_type=jax.ShapeDtypeStruct((NC * N,), jnp.float32),
    mesh=_MESH,
    compiler_params=_SC_PARAMS,
    scratch_types=(
        [pltpu.VMEM((DEG_K,), jnp.float32),
         pltpu.VMEM((N - (NS - 1) * SEG,), jnp.float32),
         pltpu.VMEM((DEG_CH, DEG_K), jnp.int32),
         pltpu.VMEM_SHARED((N,), jnp.float32)]
        + [pltpu.SemaphoreType.DMA for _ in range(NBUF)]
    ),
)(_deg_body)


SCAT_EPT = E // NS
LOOK = 3


def _scatter_body(W, K, g_hbm, edges_hbm, out_hbm, zbuf, raw, *ring):
    SCAT_CH = SCAT_EPT // K
    NSC = SCAT_CH // NBUF
    srcsc = ring[0:2]
    dstsc = ring[2:4]
    rows_v = ring[4:4 + NBUF]
    acc_sh = ring[4 + NBUF]
    gsem = ring[5 + NBUF:5 + 2 * NBUF]
    ssem = ring[5 + 2 * NBUF:5 + 3 * NBUF]
    cid = lax.axis_index("c")
    sid = lax.axis_index("s")
    off = cid * N

    _zero_buf(zbuf, ZRS, W)
    for k in range(RPT // ZRS):
        pltpu.sync_copy(zbuf, acc_sh.at[pl.ds(sid * RPT + k * ZRS, ZRS)])
    plsc.subcore_barrier()

    def load_sc(scj, p):
        base = sid * SCAT_CH + scj * NBUF
        pltpu.sync_copy(edges_hbm.at[0, pl.ds(base, NBUF)], raw)
        pltpu.sync_copy(edges_hbm.at[1, pl.ds(base, NBUF)], dstsc[p])
        for r in range(NBUF):
            for o in _offs(K):
                sl = pl.ds(o, LANES)
                srcsc[p][r, sl] = raw[r, sl] + off

    def issue_gather(set_, row, b):
        pltpu.async_copy(g_hbm.at[srcsc[set_].at[row]], rows_v[b], gsem[b])

    WT = NBUF - LOOK

    load_sc(0, 0)
    for c0 in range(LOOK):
        issue_gather(0, c0, c0)

    def body(j4, _):
        for par in range(2):
            j2 = j4 * 2 + par
            for b in range(NBUF):
                bi = (b + LOOK) % NBUF
                if b >= WT:
                    wset, wrow = par, b - WT
                else:
                    wset, wrow = 1 - par, b + LOOK

                @pl.when(j2 * NBUF + b - WT >= 0)
                def _():
                    pltpu.make_async_copy(
                        rows_v[bi], acc_sh.at[dstsc[wset].at[wrow]],
                        ssem[bi]).wait()

                if b == WT:
                    @pl.when(j2 + 1 < NSC)
                    def _():
                        load_sc(j2 + 1, 1 - par)

                if b < WT:
                    iset, irow = par, b + LOOK
                else:
                    iset, irow = 1 - par, b - WT

                @pl.when(j2 * NBUF + b + LOOK < SCAT_CH)
                def _():
                    issue_gather(iset, irow, bi)

                pltpu.make_async_copy(
                    g_hbm.at[srcsc[par].at[b]], rows_v[b], gsem[b]).wait()
                pltpu.async_copy(
                    rows_v[b], acc_sh.at[dstsc[par].at[b]], ssem[b], add=True)
        return 0

    lax.fori_loop(0, NSC // 2, body, 0)
    for c in range(SCAT_CH - NBUF + LOOK, SCAT_CH):
        b = c % NBUF
        pltpu.make_async_copy(
            rows_v[b], acc_sh.at[dstsc[(c // NBUF) % 2].at[b]], ssem[b]).wait()
    plsc.subcore_barrier()

    for k in range(RPT // ZRS):
        r0 = sid * RPT + k * ZRS
        pltpu.sync_copy(acc_sh.at[pl.ds(r0, ZRS)], zbuf)
        pltpu.sync_copy(zbuf, out_hbm.at[pl.ds(off + r0, ZRS)])


def _make_scatter(W, K):
    return functools.partial(
        pl.kernel,
        out_type=jax.ShapeDtypeStruct((NC * N, W), jnp.float32),
        mesh=_MESH,
        compiler_params=_SC_PARAMS,
        scratch_types=(
            [pltpu.VMEM((ZRS, W), jnp.float32),
             pltpu.VMEM((NBUF, K), jnp.int32)]
            + [pltpu.VMEM((NBUF, K), jnp.int32) for _ in range(4)]
            + [pltpu.VMEM((K, W), jnp.float32) for _ in range(NBUF)]
            + [pltpu.VMEM_SHARED((N, W), jnp.float32)]
            + [pltpu.SemaphoreType.DMA for _ in range(2 * NBUF)]
        ),
    )(functools.partial(_scatter_body, W, K))


K1 = 50
K2 = 100
_scatter128 = _make_scatter(H // NC, K1)
_scatter32 = _make_scatter(C // NC, K2)


def _dinv_block(deg_a, deg_b):
    return lax.rsqrt(deg_a + deg_b + 1.0)


def _mm1_body(x_ref, w_ref, dg_ref, out_ref):
    dinv = lax.rsqrt(dg_ref[0] + dg_ref[1] + 1.0)
    h = jnp.dot(x_ref[...], w_ref[...], preferred_element_type=jnp.float32)
    out_ref[0] = dinv * h[:, 0:128]
    out_ref[1] = dinv * h[:, 128:256]


_mm1 = pl.pallas_call(
    _mm1_body,
    grid=(NB,),
    in_specs=[
        pl.BlockSpec((BN, D), lambda i: (i, 0)),
        pl.BlockSpec((D, H), lambda i: (0, 0)),
        pl.BlockSpec((2, BN, 1), lambda i: (0, i, 0)),
    ],
    out_specs=pl.BlockSpec((2, BN, H // NC), lambda i: (0, i, 0)),
    out_shape=jax.ShapeDtypeStruct((NC, N, H // NC), jnp.float32),
)


def _mm2_body(s1_ref, g1_ref, w2_ref, b1_ref, dg_ref, out_ref):
    dinv = lax.rsqrt(dg_ref[0] + dg_ref[1] + 1.0)
    x1a = jax.nn.relu(dinv * (s1_ref[0] + g1_ref[0]) + b1_ref[0:128])
    x1b = jax.nn.relu(dinv * (s1_ref[1] + g1_ref[1]) + b1_ref[128:256])
    x1 = jnp.concatenate([x1a, x1b], axis=1)
    acc = jnp.dot(x1, w2_ref[...], preferred_element_type=jnp.float32)
    out_ref[0] = dinv * acc[:, 0:32]
    out_ref[1] = dinv * acc[:, 32:64]


_mm2 = pl.pallas_call(
    _mm2_body,
    grid=(NB,),
    in_specs=[
        pl.BlockSpec((2, BN, H // NC), lambda i: (0, i, 0)),
        pl.BlockSpec((2, BN, H // NC), lambda i: (0, i, 0)),
        pl.BlockSpec((H, C), lambda i: (0, 0)),
        pl.BlockSpec((H,), lambda i: (0,)),
        pl.BlockSpec((2, BN, 1), lambda i: (0, i, 0)),
    ],
    out_specs=pl.BlockSpec((2, BN, C // NC), lambda i: (0, i, 0)),
    out_shape=jax.ShapeDtypeStruct((NC, N, C // NC), jnp.float32),
)


def _fin_body(s2_ref, g2_ref, b2_ref, dg_ref, out_ref):
    dinv = lax.rsqrt(dg_ref[0] + dg_ref[1] + 1.0)
    ha = dinv * (s2_ref[0] + g2_ref[0]) + b2_ref[0:32]
    hb = dinv * (s2_ref[1] + g2_ref[1]) + b2_ref[32:64]
    out_ref[...] = jnp.concatenate([ha, hb], axis=1)


_fin = pl.pallas_call(
    _fin_body,
    grid=(NB,),
    in_specs=[
        pl.BlockSpec((2, BN, C // NC), lambda i: (0, i, 0)),
        pl.BlockSpec((2, BN, C // NC), lambda i: (0, i, 0)),
        pl.BlockSpec((C,), lambda i: (0,)),
        pl.BlockSpec((2, BN, 1), lambda i: (0, i, 0)),
    ],
    out_specs=pl.BlockSpec((BN, C), lambda i: (i, 0)),
    out_shape=jax.ShapeDtypeStruct((N, C), jnp.float32),
)


def kernel(last_e_emb, edge_index, W1, b1, W2, b2):
    edges3a = edge_index.reshape(2, E // K1, K1)
    edges3b = edge_index.reshape(2, E // K2, K2)
    deg = _deg_kernel(edges3a).reshape(NC, N, 1)
    g1 = _mm1(last_e_emb, W1, deg)
    s1 = _scatter128(g1.reshape(NC * N, H // NC), edges3a)
    s1v = s1.reshape(NC, N, H // NC)
    g2 = _mm2(s1v, g1, W2, b1, deg)
    s2 = _scatter32(g2.reshape(NC * N, C // NC), edges3b)
    return _fin(s2.reshape(NC, N, C // NC), g2, b2, deg)
